# baseline (device time: 337645 ns/iter reference)
import jax
import jax.numpy as jnp
from jax import lax
from jax.experimental import pallas as pl
from jax.experimental.pallas import tpu as pltpu


def kernel(Q, K, V):
    b, q_len, h, d = Q.shape
    k_len = K.shape[1]
    scale = d ** -0.5

    Qr = Q.reshape(b, h, d)

    def body(q_ref, k_ref, v_ref, out_ref,
             acc_o, acc_m, acc_l, recv_o, recv_m, recv_l,
             send_sems, recv_sems):
        bi = pl.program_id(0)
        mx = lax.axis_index("x")
        my = lax.axis_index("y")
        mz = lax.axis_index("z")
        partner = (mx, my, 1 - mz)

        @pl.when(bi == 0)
        def _entry_barrier():
            bar = pltpu.get_barrier_semaphore()
            pl.semaphore_signal(
                bar, inc=1, device_id=partner,
                device_id_type=pl.DeviceIdType.MESH,
            )
            pl.semaphore_wait(bar, 1)

        qb = q_ref[pl.ds(bi, 1), :, :] * scale
        st = jnp.sum(k_ref[0] * qb, axis=2)
        m = jnp.max(st, axis=0, keepdims=True)
        p = jnp.exp(st - m)
        l = jnp.sum(p, axis=0, keepdims=True)
        p3 = p.reshape(k_len, h, 1)
        ov = jnp.sum(p3 * v_ref[0], axis=0)

        acc_o[pl.ds(bi, 1), :, :] = ov[jnp.newaxis]
        acc_m[pl.ds(bi, 1), :] = m
        acc_l[pl.ds(bi, 1), :] = l

        @pl.when(bi == b - 1)
        def _exchange_and_combine():
            rdmas = []
            for i, (src, dst) in enumerate(
                    ((acc_o, recv_o), (acc_m, recv_m), (acc_l, recv_l))):
                r = pltpu.make_async_remote_copy(
                    src_ref=src, dst_ref=dst,
                    send_sem=send_sems.at[i], recv_sem=recv_sems.at[i],
                    device_id=partner,
                    device_id_type=pl.DeviceIdType.MESH,
                )
                r.start()
                rdmas.append(r)
            for r in rdmas:
                r.wait()

            ma = acc_m[...]
            mb = recv_m[...]
            mn = jnp.maximum(ma, mb)
            alpha = jnp.exp(ma - mn)
            beta = jnp.exp(mb - mn)
            ln = alpha * acc_l[...] + beta * recv_l[...]
            a3 = alpha.reshape(b, h, 1)
            b3 = beta.reshape(b, h, 1)
            out_ref[...] = (a3 * acc_o[...] + b3 * recv_o[...]) \
                / ln.reshape(b, h, 1)

    out = pl.pallas_call(
        body,
        grid=(b,),
        out_shape=jax.ShapeDtypeStruct((b, h, d), jnp.float32),
        in_specs=[
            pl.BlockSpec((b, h, d), lambda i: (0, 0, 0)),
            pl.BlockSpec((1, k_len, h, d), lambda i: (i, 0, 0, 0)),
            pl.BlockSpec((1, k_len, h, d), lambda i: (i, 0, 0, 0)),
        ],
        out_specs=pl.BlockSpec((b, h, d), lambda i: (0, 0, 0)),
        scratch_shapes=[
            pltpu.VMEM((b, h, d), jnp.float32),
            pltpu.VMEM((b, h), jnp.float32),
            pltpu.VMEM((b, h), jnp.float32),
            pltpu.VMEM((b, h, d), jnp.float32),
            pltpu.VMEM((b, h), jnp.float32),
            pltpu.VMEM((b, h), jnp.float32),
            pltpu.SemaphoreType.DMA((3,)),
            pltpu.SemaphoreType.DMA((3,)),
        ],
        compiler_params=pltpu.CompilerParams(
            dimension_semantics=("arbitrary",),
            collective_id=0,
            vmem_limit_bytes=64 * 1024 * 1024,
        ),
    )(Qr, K, V)

    return out.reshape(b, q_len, h, d)


# device time: 325487 ns/iter; 1.0374x vs baseline; 1.0374x over previous
import jax
import jax.numpy as jnp
from jax import lax
from jax.experimental import pallas as pl
from jax.experimental.pallas import tpu as pltpu


def kernel(Q, K, V):
    b, q_len, h, d = Q.shape
    k_len = K.shape[1]
    kh = k_len * h
    scale = d ** -0.5

    Kr = K.reshape(b, kh, d)
    Vr = V.reshape(b, kh, d)
    Qt = jnp.swapaxes(Q.reshape(b, h, d), 1, 2) * scale

    def body(qt_ref, k_ref, v_ref, out_ref,
             acc_o, acc_m, acc_l, recv_o, recv_m, recv_l,
             send_sems, recv_sems):
        bi = pl.program_id(0)
        mx = lax.axis_index("x")
        my = lax.axis_index("y")
        mz = lax.axis_index("z")
        partner = (mx, my, 1 - mz)

        @pl.when(bi == 0)
        def _entry_barrier():
            bar = pltpu.get_barrier_semaphore()
            pl.semaphore_signal(
                bar, inc=1, device_id=partner,
                device_id_type=pl.DeviceIdType.MESH,
            )
            pl.semaphore_wait(bar, 1)

        qt = qt_ref[bi].astype(jnp.bfloat16)
        k2 = k_ref[0].astype(jnp.bfloat16)
        g = lax.dot_general(
            k2, qt, (((1,), (0,)), ((), ())),
            preferred_element_type=jnp.float32,
        )
        rowm = (lax.broadcasted_iota(jnp.int32, (kh, h), 0) % h) \
            == lax.broadcasted_iota(jnp.int32, (kh, h), 1)
        gm = jnp.where(rowm, g, -1e30)
        m = jnp.max(gm, axis=0, keepdims=True)
        pex = jnp.exp(gm - m)
        l = jnp.sum(pex, axis=0, keepdims=True)
        ov = lax.dot_general(
            pex.astype(jnp.bfloat16), v_ref[0].astype(jnp.bfloat16),
            (((0,), (0,)), ((), ())),
            preferred_element_type=jnp.float32,
        )

        acc_o[pl.ds(bi, 1), :, :] = ov[jnp.newaxis]
        acc_m[pl.ds(bi, 1), :] = m
        acc_l[pl.ds(bi, 1), :] = l

        @pl.when(bi == b - 1)
        def _exchange_and_combine():
            rdmas = []
            for i, (src, dst) in enumerate(
                    ((acc_o, recv_o), (acc_m, recv_m), (acc_l, recv_l))):
                r = pltpu.make_async_remote_copy(
                    src_ref=src, dst_ref=dst,
                    send_sem=send_sems.at[i], recv_sem=recv_sems.at[i],
                    device_id=partner,
                    device_id_type=pl.DeviceIdType.MESH,
                )
                r.start()
                rdmas.append(r)
            for r in rdmas:
                r.wait()

            ma = acc_m[...]
            mb = recv_m[...]
            mn = jnp.maximum(ma, mb)
            alpha = jnp.exp(ma - mn)
            beta = jnp.exp(mb - mn)
            ln = alpha * acc_l[...] + beta * recv_l[...]
            a3 = alpha.reshape(b, h, 1)
            b3 = beta.reshape(b, h, 1)
            out_ref[...] = (a3 * acc_o[...] + b3 * recv_o[...]) \
                / ln.reshape(b, h, 1)

    out = pl.pallas_call(
        body,
        grid=(b,),
        out_shape=jax.ShapeDtypeStruct((b, h, d), jnp.float32),
        in_specs=[
            pl.BlockSpec((b, d, h), lambda i: (0, 0, 0)),
            pl.BlockSpec((1, kh, d), lambda i: (i, 0, 0)),
            pl.BlockSpec((1, kh, d), lambda i: (i, 0, 0)),
        ],
        out_specs=pl.BlockSpec((b, h, d), lambda i: (0, 0, 0)),
        scratch_shapes=[
            pltpu.VMEM((b, h, d), jnp.float32),
            pltpu.VMEM((b, h), jnp.float32),
            pltpu.VMEM((b, h), jnp.float32),
            pltpu.VMEM((b, h, d), jnp.float32),
            pltpu.VMEM((b, h), jnp.float32),
            pltpu.VMEM((b, h), jnp.float32),
            pltpu.SemaphoreType.DMA((3,)),
            pltpu.SemaphoreType.DMA((3,)),
        ],
        compiler_params=pltpu.CompilerParams(
            dimension_semantics=("arbitrary",),
            collective_id=0,
            vmem_limit_bytes=100 * 1024 * 1024,
        ),
    )(Qt, Kr, Vr)

    return out.reshape(b, q_len, h, d)


# device time: 324529 ns/iter; 1.0404x vs baseline; 1.0030x over previous
import jax
import jax.numpy as jnp
from jax import lax
from jax.experimental import pallas as pl
from jax.experimental.pallas import tpu as pltpu


def kernel(Q, K, V):
    b, q_len, h, d = Q.shape
    k_len = K.shape[1]
    kh = k_len * h
    scale = d ** -0.5

    Qt = jnp.swapaxes(Q.reshape(b, h, d), 1, 2) * scale

    def body(qt_ref, k_ref, v_ref, out_ref,
             acc_o, acc_m, acc_l, recv_o, recv_m, recv_l,
             send_sems, recv_sems):
        bi = pl.program_id(0)
        mx = lax.axis_index("x")
        my = lax.axis_index("y")
        mz = lax.axis_index("z")
        partner = (mx, my, 1 - mz)

        @pl.when(bi == 0)
        def _entry_barrier():
            bar = pltpu.get_barrier_semaphore()
            pl.semaphore_signal(
                bar, inc=1, device_id=partner,
                device_id_type=pl.DeviceIdType.MESH,
            )
            pl.semaphore_wait(bar, 1)

        qt = qt_ref[bi].astype(jnp.bfloat16)
        k2 = k_ref[0].reshape(kh, d).astype(jnp.bfloat16)
        g = lax.dot_general(
            k2, qt, (((1,), (0,)), ((), ())),
            preferred_element_type=jnp.float32,
        )
        rowm = (lax.broadcasted_iota(jnp.int32, (kh, h), 0) % h) \
            == lax.broadcasted_iota(jnp.int32, (kh, h), 1)
        gm = jnp.where(rowm, g, -1e30)
        m = jnp.max(gm, axis=0, keepdims=True)
        pex = jnp.exp(gm - m)
        l = jnp.sum(pex, axis=0, keepdims=True)
        v2 = v_ref[0].reshape(kh, d).astype(jnp.bfloat16)
        ov = lax.dot_general(
            pex.astype(jnp.bfloat16), v2,
            (((0,), (0,)), ((), ())),
            preferred_element_type=jnp.float32,
        )

        acc_o[pl.ds(bi, 1), :, :] = ov[jnp.newaxis]
        acc_m[pl.ds(bi, 1), :] = m
        acc_l[pl.ds(bi, 1), :] = l

        @pl.when(bi == b - 1)
        def _exchange_and_combine():
            rdmas = []
            for i, (src, dst) in enumerate(
                    ((acc_o, recv_o), (acc_m, recv_m), (acc_l, recv_l))):
                r = pltpu.make_async_remote_copy(
                    src_ref=src, dst_ref=dst,
                    send_sem=send_sems.at[i], recv_sem=recv_sems.at[i],
                    device_id=partner,
                    device_id_type=pl.DeviceIdType.MESH,
                )
                r.start()
                rdmas.append(r)
            for r in rdmas:
                r.wait()

            ma = acc_m[...]
            mb = recv_m[...]
            mn = jnp.maximum(ma, mb)
            alpha = jnp.exp(ma - mn)
            beta = jnp.exp(mb - mn)
            ln = alpha * acc_l[...] + beta * recv_l[...]
            a3 = alpha.reshape(b, h, 1)
            b3 = beta.reshape(b, h, 1)
            out_ref[...] = (a3 * acc_o[...] + b3 * recv_o[...]) \
                / ln.reshape(b, h, 1)

    out = pl.pallas_call(
        body,
        grid=(b,),
        out_shape=jax.ShapeDtypeStruct((b, h, d), jnp.float32),
        in_specs=[
            pl.BlockSpec((b, d, h), lambda i: (0, 0, 0)),
            pl.BlockSpec((1, k_len, h, d), lambda i: (i, 0, 0, 0)),
            pl.BlockSpec((1, k_len, h, d), lambda i: (i, 0, 0, 0)),
        ],
        out_specs=pl.BlockSpec((b, h, d), lambda i: (0, 0, 0)),
        scratch_shapes=[
            pltpu.VMEM((b, h, d), jnp.float32),
            pltpu.VMEM((b, h), jnp.float32),
            pltpu.VMEM((b, h), jnp.float32),
            pltpu.VMEM((b, h, d), jnp.float32),
            pltpu.VMEM((b, h), jnp.float32),
            pltpu.VMEM((b, h), jnp.float32),
            pltpu.SemaphoreType.DMA((3,)),
            pltpu.SemaphoreType.DMA((3,)),
        ],
        compiler_params=pltpu.CompilerParams(
            dimension_semantics=("arbitrary",),
            collective_id=0,
            vmem_limit_bytes=100 * 1024 * 1024,
        ),
    )(Qt, K, V)

    return out.reshape(b, q_len, h, d)


# device time: 208126 ns/iter; 1.6223x vs baseline; 1.5593x over previous
import jax
import jax.numpy as jnp
from jax import lax
from jax.experimental import pallas as pl
from jax.experimental.pallas import tpu as pltpu


def kernel(Q, K, V):
    b, q_len, h, d = Q.shape
    k_len = K.shape[1]
    hd = h * d
    scale = d ** -0.5

    Kf = K.astype(jnp.bfloat16).reshape(b, k_len, hd)
    Vf = V.astype(jnp.bfloat16).reshape(b, k_len, hd)
    Qf = (Q * scale).reshape(b, hd, 1)

    def body(q_ref, k_ref, v_ref, out_ref, acc_ref, recv_ref, send_sem, recv_sem):
        bi = pl.program_id(0)
        mx = lax.axis_index("x")
        my = lax.axis_index("y")
        mz = lax.axis_index("z")
        partner = (mx, my, 1 - mz)

        @pl.when(bi == 0)
        def _entry_barrier():
            bar = pltpu.get_barrier_semaphore()
            pl.semaphore_signal(
                bar, inc=1, device_id=partner,
                device_id_type=pl.DeviceIdType.MESH,
            )
            pl.semaphore_wait(bar, 1)

        subj = lax.broadcasted_iota(jnp.int32, (hd, h), 0)
        laneh = lax.broadcasted_iota(jnp.int32, (hd, h), 1)
        maskjh = (subj // d) == laneh
        subh = lax.broadcasted_iota(jnp.int32, (h, hd), 0)
        lanej = lax.broadcasted_iota(jnp.int32, (h, hd), 1)
        maskhj = (lanej // d) == subh

        qcol = q_ref[bi]
        qbd = jnp.where(
            maskjh, jnp.broadcast_to(qcol, (hd, h)), 0.0
        ).astype(jnp.bfloat16)
        kb = k_ref[0]
        vb = v_ref[0]

        st = lax.dot_general(
            kb, qbd, (((1,), (0,)), ((), ())),
            preferred_element_type=jnp.float32,
        )
        m = jnp.max(st, axis=0, keepdims=True)
        p = jnp.exp(st - m)
        l = jnp.sum(p, axis=0, keepdims=True)

        g = lax.dot_general(
            p.astype(jnp.bfloat16), vb, (((0,), (0,)), ((), ())),
            preferred_element_type=jnp.float32,
        )
        o_flat = jnp.sum(jnp.where(maskhj, g, 0.0), axis=0, keepdims=True)
        spread = maskhj.astype(jnp.float32)
        m_flat = lax.dot_general(
            m, spread, (((1,), (0,)), ((), ())),
            preferred_element_type=jnp.float32,
        )
        l_flat = lax.dot_general(
            l, spread, (((1,), (0,)), ((), ())),
            preferred_element_type=jnp.float32,
        )

        acc_ref[pl.ds(bi, 1), :] = o_flat
        acc_ref[pl.ds(b + bi, 1), :] = m_flat
        acc_ref[pl.ds(2 * b + bi, 1), :] = l_flat

        @pl.when(bi == b - 1)
        def _exchange_and_combine():
            rdma = pltpu.make_async_remote_copy(
                src_ref=acc_ref,
                dst_ref=recv_ref,
                send_sem=send_sem,
                recv_sem=recv_sem,
                device_id=partner,
                device_id_type=pl.DeviceIdType.MESH,
            )
            rdma.start()
            rdma.wait()

            oa = acc_ref[0:b, :]
            ma = acc_ref[b:2 * b, :]
            la = acc_ref[2 * b:3 * b, :]
            ob = recv_ref[0:b, :]
            mb = recv_ref[b:2 * b, :]
            lb = recv_ref[2 * b:3 * b, :]
            mn = jnp.maximum(ma, mb)
            alpha = jnp.exp(ma - mn)
            beta = jnp.exp(mb - mn)
            out_ref[...] = (alpha * oa + beta * ob) / (alpha * la + beta * lb)

    out = pl.pallas_call(
        body,
        grid=(b,),
        out_shape=jax.ShapeDtypeStruct((b, hd), jnp.float32),
        in_specs=[
            pl.BlockSpec((b, hd, 1), lambda i: (0, 0, 0)),
            pl.BlockSpec((1, k_len, hd), lambda i: (i, 0, 0)),
            pl.BlockSpec((1, k_len, hd), lambda i: (i, 0, 0)),
        ],
        out_specs=pl.BlockSpec((b, hd), lambda i: (0, 0)),
        scratch_shapes=[
            pltpu.VMEM((3 * b, hd), jnp.float32),
            pltpu.VMEM((3 * b, hd), jnp.float32),
            pltpu.SemaphoreType.DMA,
            pltpu.SemaphoreType.DMA,
        ],
        compiler_params=pltpu.CompilerParams(
            dimension_semantics=("arbitrary",),
            collective_id=0,
            vmem_limit_bytes=64 * 1024 * 1024,
        ),
    )(Qf, Kf, Vf)

    return out.reshape(b, q_len, h, d)


# device time: 73169 ns/iter; 4.6146x vs baseline; 2.8445x over previous
import jax
import jax.numpy as jnp
from jax import lax
from jax.experimental import pallas as pl
from jax.experimental.pallas import tpu as pltpu

N_RING = 4


def kernel(Q, K, V):
    b, q_len, h, d = Q.shape
    k_len = K.shape[1]
    hd = h * d
    nb = b // N_RING
    scale = d ** -0.5

    mx = lax.axis_index("x")
    my = lax.axis_index("y")
    r = 2 * mx + jnp.where(mx == 1, 1 - my, my)

    Kq = lax.dynamic_slice_in_dim(K, r * nb, nb, axis=0)
    Vq = lax.dynamic_slice_in_dim(V, r * nb, nb, axis=0)
    Kf = Kq.reshape(nb, k_len, hd)
    Vf = Vq.reshape(nb, k_len, hd)
    Qf = (Q * scale).reshape(b, hd)

    def ring_coords(pos):
        x = jnp.where(pos >= 2, 1, 0)
        y = jnp.where((pos == 1) | (pos == 2), 1, 0)
        return x, y

    def body(q_ref, k_ref, v_ref, out_ref,
             acc_ref, recv_ref, comm_ref,
             z_send, z_recv, ring_send, ring_recv):
        bi = pl.program_id(0)
        mx = lax.axis_index("x")
        my = lax.axis_index("y")
        mz = lax.axis_index("z")
        partner = (mx, my, 1 - mz)
        r = 2 * mx + jnp.where(mx == 1, 1 - my, my)
        lx, ly = ring_coords(lax.rem(r + 3, 4))
        rx, ry = ring_coords(lax.rem(r + 1, 4))

        @pl.when(bi == 0)
        def _entry_barrier():
            bar = pltpu.get_barrier_semaphore()
            for dev in (partner, (lx, ly, mz), (rx, ry, mz)):
                pl.semaphore_signal(
                    bar, inc=1, device_id=dev,
                    device_id_type=pl.DeviceIdType.MESH,
                )
            pl.semaphore_wait(bar, 3)

        lane = lax.broadcasted_iota(jnp.int32, (h, hd), 1)
        sub = lax.broadcasted_iota(jnp.int32, (h, hd), 0)
        mask = (lane // d) == sub

        qrow = q_ref[pl.ds(r * nb + bi, 1), :]
        qexp = jnp.where(mask, jnp.broadcast_to(qrow, (h, hd)), 0.0)
        kb = k_ref[0]
        vb = v_ref[0]

        s = lax.dot_general(
            qexp, kb, (((1,), (1,)), ((), ())),
            preferred_element_type=jnp.float32,
        )
        m = jnp.max(s, axis=1, keepdims=True)
        p = jnp.exp(s - m)
        l = jnp.sum(p, axis=1, keepdims=True)

        g = lax.dot_general(
            p, vb, (((1,), (0,)), ((), ())),
            preferred_element_type=jnp.float32,
        )
        o_flat = jnp.sum(jnp.where(mask, g, 0.0), axis=0, keepdims=True)
        m_flat = jnp.sum(
            jnp.where(mask, jnp.broadcast_to(m, (h, hd)), 0.0),
            axis=0, keepdims=True)
        l_flat = jnp.sum(
            jnp.where(mask, jnp.broadcast_to(l, (h, hd)), 0.0),
            axis=0, keepdims=True)

        acc_ref[pl.ds(bi, 1), :] = o_flat
        acc_ref[pl.ds(nb + bi, 1), :] = m_flat
        acc_ref[pl.ds(2 * nb + bi, 1), :] = l_flat

        @pl.when(bi == nb - 1)
        def _exchange_combine_gather():
            rdma = pltpu.make_async_remote_copy(
                src_ref=acc_ref, dst_ref=recv_ref,
                send_sem=z_send, recv_sem=z_recv,
                device_id=partner,
                device_id_type=pl.DeviceIdType.MESH,
            )
            rdma.start()
            rdma.wait()

            ma = acc_ref[nb:2 * nb, :]
            mb = recv_ref[nb:2 * nb, :]
            mn = jnp.maximum(ma, mb)
            alpha = jnp.exp(ma - mn)
            beta = jnp.exp(mb - mn)
            oc = (alpha * acc_ref[0:nb, :] + beta * recv_ref[0:nb, :]) \
                / (alpha * acc_ref[2 * nb:3 * nb, :]
                   + beta * recv_ref[2 * nb:3 * nb, :])

            out_ref[pl.ds(r, 1), :, :] = oc[jnp.newaxis]
            comm_ref[0, :, :] = oc

            for hop in range(N_RING - 1):
                send_slot = hop % 2
                recv_slot = (hop + 1) % 2
                ring = pltpu.make_async_remote_copy(
                    src_ref=comm_ref.at[send_slot],
                    dst_ref=comm_ref.at[recv_slot],
                    send_sem=ring_send.at[send_slot],
                    recv_sem=ring_recv.at[recv_slot],
                    device_id=(rx, ry, mz),
                    device_id_type=pl.DeviceIdType.MESH,
                )
                ring.start()
                ring.wait()
                origin = lax.rem(r + 3 - hop, 4)
                out_ref[pl.ds(origin, 1), :, :] = comm_ref[recv_slot][jnp.newaxis]

    out = pl.pallas_call(
        body,
        grid=(nb,),
        out_shape=jax.ShapeDtypeStruct((N_RING, nb, hd), jnp.float32),
        in_specs=[
            pl.BlockSpec((b, hd), lambda i: (0, 0)),
            pl.BlockSpec((1, k_len, hd), lambda i: (i, 0, 0)),
            pl.BlockSpec((1, k_len, hd), lambda i: (i, 0, 0)),
        ],
        out_specs=pl.BlockSpec((N_RING, nb, hd), lambda i: (0, 0, 0)),
        scratch_shapes=[
            pltpu.VMEM((3 * nb, hd), jnp.float32),
            pltpu.VMEM((3 * nb, hd), jnp.float32),
            pltpu.VMEM((2, nb, hd), jnp.float32),
            pltpu.SemaphoreType.DMA,
            pltpu.SemaphoreType.DMA,
            pltpu.SemaphoreType.DMA((2,)),
            pltpu.SemaphoreType.DMA((2,)),
        ],
        compiler_params=pltpu.CompilerParams(
            dimension_semantics=("arbitrary",),
            collective_id=0,
        ),
    )(Qf, Kf, Vf)

    return out.reshape(b, q_len, h, d)
